# baseline (speedup 1.0000x reference)
import functools

import jax
import jax.numpy as jnp
from jax import lax
from jax.experimental import pallas as pl
from jax.experimental.pallas import tpu as pltpu
from jax.experimental.pallas import tpu_sc as plsc

TOPK = 15
W_TILE = 2048
INT_MAX = 2**31 - 1


def _make_sc_gather(n_rows, table_shape, dtype):
    V, D = table_shape
    info = plsc.get_sparse_core_info()
    NC, NS = info.num_cores, info.num_subcores
    NW = NC * NS
    assert n_rows % NW == 0
    b_per_w = n_rows // NW
    mesh = plsc.VectorSubcoreMesh(core_axis_name="c", subcore_axis_name="s")

    @functools.partial(
        pl.kernel,
        mesh=mesh,
        out_type=jax.ShapeDtypeStruct((n_rows, D), dtype),
        scratch_types=[
            pltpu.VMEM((b_per_w,), jnp.int32),
            pltpu.VMEM((b_per_w, D), dtype),
            pltpu.SemaphoreType.DMA,
        ],
    )
    def gather(table_hbm, idx_hbm, out_hbm, idx_v, rows_v, sem):
        wid = lax.axis_index("s") * NC + lax.axis_index("c")
        base = wid * b_per_w
        pltpu.sync_copy(idx_hbm.at[pl.ds(base, b_per_w)], idx_v)
        cps = []
        for j in range(b_per_w):
            if j % 16 == 0:
                chunk = idx_v[pl.ds(j, 16)]
            r = chunk[j % 16]
            cps.append(pltpu.async_copy(table_hbm.at[pl.ds(r, 1)],
                                        rows_v.at[pl.ds(j, 1)], sem))
        for cp in cps:
            cp.wait()
        pltpu.sync_copy(rows_v, out_hbm.at[pl.ds(base, b_per_w)])

    return gather


def _score_body(ent_ref, att_ref, ctx_ref, max_ref):
    es = ent_ref[...] * att_ref[...]
    c = ctx_ref[...]
    s = lax.dot_general(es, c, (((1,), (1,)), ((), ())),
                        preferred_element_type=jnp.float32,
                        precision=lax.Precision.DEFAULT)
    max_ref[...] = jnp.max(s, axis=0)[None, :]


def _make_tc_score(n_ent, n_words, d, w_tile):
    n_tiles = n_words // w_tile
    return pl.pallas_call(
        _score_body,
        grid=(n_tiles,),
        in_specs=[
            pl.BlockSpec((n_ent, d), lambda i: (0, 0)),
            pl.BlockSpec((1, d), lambda i: (0, 0)),
            pl.BlockSpec((w_tile, d), lambda i: (i, 0)),
        ],
        out_specs=pl.BlockSpec((1, w_tile), lambda i: (0, i)),
        out_shape=jax.ShapeDtypeStruct((1, n_words), jnp.float32),
        compiler_params=pltpu.CompilerParams(
            dimension_semantics=("parallel",)),
    )


def _finish_body(sc_ref, ent_ref, tok_ref, ctx_lo_ref, ctx_hi_ref, out_ref,
                 rowbuf_ref, sem, *, w_tile, k, half):
    sc = sc_ref[...]
    flat = lax.broadcasted_iota(jnp.int32, sc.shape, 1)
    vals, ids = [], []
    cur = sc
    for _ in range(k):
        m = jnp.max(cur)
        idx = jnp.min(jnp.where(cur == m, flat, INT_MAX))
        vals.append(m)
        ids.append(idx)
        cur = jnp.where(flat == idx, -jnp.inf, cur)

    lane = lax.broadcasted_iota(jnp.int32, (1, 128), 1)
    vv = jnp.full((1, 128), -jnp.inf, dtype=jnp.float32)
    for j in range(k):
        vv = jnp.where(lane == j, vals[j], vv)
    e = jnp.exp(vv - vals[0])
    p = e / jnp.sum(e)
    p16 = p[:, :16]

    rowbuf_ref[pl.ds(k, 1), :] = jnp.zeros((1, rowbuf_ref.shape[1]),
                                           jnp.float32)
    for j in range(k):
        @pl.when(ids[j] < half)
        def _lo():
            pltpu.make_async_copy(ctx_lo_ref.at[pl.ds(ids[j], 1)],
                                  rowbuf_ref.at[pl.ds(j, 1)], sem).start()

        @pl.when(ids[j] >= half)
        def _hi():
            pltpu.make_async_copy(ctx_hi_ref.at[pl.ds(ids[j] - half, 1)],
                                  rowbuf_ref.at[pl.ds(j, 1)], sem).start()
    for j in range(k):
        @pl.when(ids[j] < half)
        def _wlo():
            pltpu.make_async_copy(ctx_lo_ref.at[pl.ds(ids[j], 1)],
                                  rowbuf_ref.at[pl.ds(j, 1)], sem).wait()

        @pl.when(ids[j] >= half)
        def _whi():
            pltpu.make_async_copy(ctx_hi_ref.at[pl.ds(ids[j] - half, 1)],
                                  rowbuf_ref.at[pl.ds(j, 1)], sem).wait()

    ctxv = lax.dot_general(p16, rowbuf_ref[...], (((1,), (0,)), ((), ())),
                           preferred_element_type=jnp.float32,
                           precision=lax.Precision.HIGHEST)
    ctxv = ctxv * tok_ref[...]
    out_ref[...] = lax.dot_general(ctxv, ent_ref[...],
                                   (((1,), (1,)), ((), ())),
                                   preferred_element_type=jnp.float32,
                                   precision=lax.Precision.HIGHEST)


def _make_tc_finish(n_ent, n_words, d, w_tile, k, half):
    return pl.pallas_call(
        functools.partial(_finish_body, w_tile=w_tile, k=k, half=half),
        in_specs=[
            pl.BlockSpec((1, n_words), lambda: (0, 0)),
            pl.BlockSpec((n_ent, d), lambda: (0, 0)),
            pl.BlockSpec((1, d), lambda: (0, 0)),
            pl.BlockSpec(memory_space=pl.ANY),
            pl.BlockSpec(memory_space=pl.ANY),
        ],
        out_specs=pl.BlockSpec((1, n_ent), lambda: (0, 0)),
        out_shape=jax.ShapeDtypeStruct((1, n_ent), jnp.float32),
        scratch_shapes=[
            pltpu.VMEM((k + 1, d), jnp.float32),
            pltpu.SemaphoreType.DMA,
        ],
    )


def kernel(candidates, context_embed, entity_table, att_mat_diag,
           tok_score_mat_diag):
    b, n_words, d = context_embed.shape
    n_ent = candidates.shape[0]
    k = min(TOPK, n_words)
    ctx = context_embed.reshape(n_words, d)
    half = n_words // 2
    ctx_lo = ctx[:half]
    ctx_hi = ctx[half:]
    gather = _make_sc_gather(n_ent, entity_table.shape, entity_table.dtype)
    ent = gather(entity_table, candidates.astype(jnp.int32))
    score = _make_tc_score(n_ent, half, d, W_TILE)
    att2 = att_mat_diag.reshape(1, d)
    max_lo = score(ent, att2, ctx_lo)
    max_hi = score(ent, att2, ctx_hi)
    maxima = jnp.concatenate([max_lo, max_hi], axis=1)
    finish = _make_tc_finish(n_ent, n_words, d, W_TILE, k, half)
    out = finish(maxima, ent, tok_score_mat_diag.reshape(1, d),
                 ctx_lo, ctx_hi)
    return out


# speedup vs baseline: 1.2141x; 1.2141x over previous
import functools

import jax
import jax.numpy as jnp
from jax import lax
from jax.experimental import pallas as pl
from jax.experimental.pallas import tpu as pltpu
from jax.experimental.pallas import tpu_sc as plsc

TOPK = 15
W_TILE = 2048
INT_MAX = 2**31 - 1


def _make_sc_gather(n_rows, table_shape, dtype):
    V, D = table_shape
    info = plsc.get_sparse_core_info()
    NC, NS = info.num_cores, info.num_subcores
    NW = NC * NS
    assert n_rows % NW == 0
    b_per_w = n_rows // NW
    mesh = plsc.VectorSubcoreMesh(core_axis_name="c", subcore_axis_name="s")

    @functools.partial(
        pl.kernel,
        mesh=mesh,
        out_type=jax.ShapeDtypeStruct((n_rows, D), dtype),
        scratch_types=[
            pltpu.VMEM((b_per_w,), jnp.int32),
            pltpu.VMEM((b_per_w, D), dtype),
            pltpu.SemaphoreType.DMA,
        ],
    )
    def gather(table_hbm, idx_hbm, out_hbm, idx_v, rows_v, sem):
        wid = lax.axis_index("s") * NC + lax.axis_index("c")
        base = wid * b_per_w
        pltpu.sync_copy(idx_hbm.at[pl.ds(base, b_per_w)], idx_v)
        cps = []
        for j in range(b_per_w):
            if j % 16 == 0:
                chunk = idx_v[pl.ds(j, 16)]
            r = chunk[j % 16]
            cps.append(pltpu.async_copy(table_hbm.at[pl.ds(r, 1)],
                                        rows_v.at[pl.ds(j, 1)], sem))
        for cp in cps:
            cp.wait()
        pltpu.sync_copy(rows_v, out_hbm.at[pl.ds(base, b_per_w)])

    return gather


def _tc_body(ent_ref, att_ref, tok_ref, ctx_ref, ctx_hbm_ref, out_ref,
             scores_ref, rowbuf_ref, sem, *, n_tiles, w_tile, k):
    i = pl.program_id(0)
    es = ent_ref[...] * att_ref[...]
    c = ctx_ref[...]
    s = lax.dot_general(es, c, (((1,), (1,)), ((), ())),
                        preferred_element_type=jnp.float32,
                        precision=lax.Precision.DEFAULT)
    scores_ref[pl.ds(i, 1), :] = jnp.max(s, axis=0)[None, :]

    @pl.when(i == n_tiles - 1)
    def _finish():
        sc = scores_ref[...]
        flat = (lax.broadcasted_iota(jnp.int32, sc.shape, 0) * w_tile
                + lax.broadcasted_iota(jnp.int32, sc.shape, 1))
        vals, ids = [], []
        cur = sc
        for _ in range(k):
            m = jnp.max(cur)
            idx = jnp.min(jnp.where(cur == m, flat, INT_MAX))
            vals.append(m)
            ids.append(idx)
            cur = jnp.where(flat == idx, -jnp.inf, cur)

        lane = lax.broadcasted_iota(jnp.int32, (1, 128), 1)
        vv = jnp.full((1, 128), -jnp.inf, dtype=jnp.float32)
        for j in range(k):
            vv = jnp.where(lane == j, vals[j], vv)
        e = jnp.exp(vv - vals[0])
        p = e / jnp.sum(e)
        p16 = p[:, :16]

        rowbuf_ref[pl.ds(k, 1), :] = jnp.zeros((1, rowbuf_ref.shape[1]),
                                               jnp.float32)
        copies = [
            pltpu.make_async_copy(ctx_hbm_ref.at[pl.ds(ids[j], 1)],
                                  rowbuf_ref.at[pl.ds(j, 1)], sem)
            for j in range(k)
        ]
        for cp in copies:
            cp.start()
        for cp in copies:
            cp.wait()

        ctxv = lax.dot_general(p16, rowbuf_ref[...], (((1,), (0,)), ((), ())),
                               preferred_element_type=jnp.float32,
                               precision=lax.Precision.HIGHEST)
        ctxv = ctxv * tok_ref[...]
        out_ref[...] = lax.dot_general(ctxv, ent_ref[...],
                                       (((1,), (1,)), ((), ())),
                                       preferred_element_type=jnp.float32,
                                       precision=lax.Precision.HIGHEST)


def _make_tc_main(n_ent, n_words, d, w_tile, k):
    n_tiles = n_words // w_tile
    return pl.pallas_call(
        functools.partial(_tc_body, n_tiles=n_tiles, w_tile=w_tile, k=k),
        grid=(n_tiles,),
        in_specs=[
            pl.BlockSpec((n_ent, d), lambda i: (0, 0)),
            pl.BlockSpec((1, d), lambda i: (0, 0)),
            pl.BlockSpec((1, d), lambda i: (0, 0)),
            pl.BlockSpec((w_tile, d), lambda i: (i, 0)),
            pl.BlockSpec(memory_space=pl.ANY),
        ],
        out_specs=pl.BlockSpec((1, n_ent), lambda i: (0, 0)),
        out_shape=jax.ShapeDtypeStruct((1, n_ent), jnp.float32),
        scratch_shapes=[
            pltpu.VMEM((n_tiles, w_tile), jnp.float32),
            pltpu.VMEM((k + 1, d), jnp.float32),
            pltpu.SemaphoreType.DMA,
        ],
        compiler_params=pltpu.CompilerParams(
            dimension_semantics=("arbitrary",)),
    )


def kernel(candidates, context_embed, entity_table, att_mat_diag,
           tok_score_mat_diag):
    b, n_words, d = context_embed.shape
    n_ent = candidates.shape[0]
    k = min(TOPK, n_words)
    ctx = context_embed.reshape(n_words, d)
    gather = _make_sc_gather(n_ent, entity_table.shape, entity_table.dtype)
    ent = gather(entity_table, candidates.astype(jnp.int32))
    main = _make_tc_main(n_ent, n_words, d, W_TILE, k)
    out = main(ent, att_mat_diag.reshape(1, d), tok_score_mat_diag.reshape(1, d),
               ctx, ctx)
    return out
